# Initial kernel scaffold; baseline (speedup 1.0000x reference)
import jax
import jax.numpy as jnp
from jax.experimental import pallas as pl


def kernel(ent_emb, attr_emb, rel_emb, rgcn_weight, dec_W, dec_b, time_emb, edge_index, edge_type, batch_data):
    raise NotImplementedError("write your pallas kernel here")


# speedup vs baseline: 5.4047x; 5.4047x over previous
import functools
import math

import jax
import jax.numpy as jnp
from jax import lax
from jax.experimental import pallas as pl
from jax.experimental.pallas import tpu as pltpu
from jax.experimental.pallas import tpu_sc as plsc

NUM_ENT = 8000
NUM_ENT_PAD = 8192
N_NODES = 10000
HIDDEN = 200
NUM_TYPES = 16
E = 320000
BATCH = 1024
W_A = 112
W_B = 96
DEG_COL = 88
RRELU_SLOPE = (1.0 / 8.0 + 1.0 / 3.0) / 2.0

IDX_ROWS = 2500
CH_ROWS = 10
PER_CORE = 125
ZS = 624
ZS_LAST = 640


def _gidx_body(src_ref, et_ref, out_ref):
    out_ref[...] = et_ref[...] * N_NODES + src_ref[...]


def _build_gidx(src2, et2):
    return pl.pallas_call(
        _gidx_body,
        out_shape=jax.ShapeDtypeStruct((IDX_ROWS, 128), jnp.int32),
    )(src2, et2)


_HT_BLK = 1000


def _ht_body(h_ref, w_ref, out_ref):
    out_ref[0] = h_ref[...] * w_ref[0]


def _build_ht(h_pad, w_pad, width):
    nb = N_NODES // _HT_BLK
    ht = pl.pallas_call(
        _ht_body,
        grid=(nb, NUM_TYPES),
        in_specs=[
            pl.BlockSpec((_HT_BLK, width), lambda b, t: (b, 0)),
            pl.BlockSpec((1, 1, width), lambda b, t: (t, 0, 0)),
        ],
        out_specs=pl.BlockSpec((1, _HT_BLK, width), lambda b, t: (t, b, 0)),
        out_shape=jax.ShapeDtypeStruct((NUM_TYPES, N_NODES, width),
                                       jnp.float32),
    )(h_pad, w_pad.reshape(NUM_TYPES, 1, width))
    return ht.reshape(NUM_TYPES * N_NODES, width)


def _sc_agg_body(ht_hbm, gidx_hbm, dst_hbm, z_hbm, out_a, out_b,
                 gbuf, dbuf, rows_a, rows_b, acc, sem_a, sem_b):
    c = lax.axis_index("c")
    s = lax.axis_index("s")

    @pl.when(s < 15)
    def _():
        pltpu.sync_copy(z_hbm.at[pl.ds(0, ZS)], acc.at[pl.ds(s * ZS, ZS)])

    @pl.when(s == 15)
    def _():
        pltpu.sync_copy(z_hbm, acc.at[pl.ds(15 * ZS, ZS_LAST)])

    plsc.subcore_barrier()

    base = c * PER_CORE + jnp.minimum(s, 13) * 8 + jnp.maximum(s - 13, 0) * 7
    cnt = jnp.where(s < 13, 8, 7)

    def body(i, carry):
        row0 = (base + i) * CH_ROWS
        pltpu.sync_copy(gidx_hbm.at[pl.ds(row0, CH_ROWS)], gbuf)
        pltpu.sync_copy(dst_hbm.at[pl.ds(row0, CH_ROWS)], dbuf)
        pltpu.async_copy(ht_hbm.at[gbuf.at[0]], rows_a, sem_a)
        for j in range(CH_ROWS):
            cur, csem = (rows_a, sem_a) if j % 2 == 0 else (rows_b, sem_b)
            nxt, nsem = (rows_b, sem_b) if j % 2 == 0 else (rows_a, sem_a)
            pltpu.make_async_copy(ht_hbm.at[gbuf.at[j]], cur, csem).wait()
            if j + 1 < CH_ROWS:
                pltpu.async_copy(ht_hbm.at[gbuf.at[j + 1]], nxt, nsem)
            pltpu.sync_copy(cur, acc.at[dbuf.at[j]], add=True)
        return carry

    lax.fori_loop(0, cnt, body, 0)
    plsc.subcore_barrier()

    out = [out_a, out_b]
    for ci in range(2):
        @pl.when(jnp.logical_and(c == ci, s < 15))
        def _(ci=ci):
            sl = pl.ds(s * ZS, ZS)
            pltpu.sync_copy(acc.at[sl], out[ci].at[sl])

        @pl.when(jnp.logical_and(c == ci, s == 15))
        def _(ci=ci):
            sl = pl.ds(15 * ZS, ZS_LAST)
            pltpu.sync_copy(acc.at[sl], out[ci].at[sl])


def _sc_agg(ht, gidx2, dst2, zeros_slab, width):
    mesh = plsc.VectorSubcoreMesh(core_axis_name="c", subcore_axis_name="s")
    k = functools.partial(
        pl.kernel,
        out_type=(
            jax.ShapeDtypeStruct((N_NODES, width), jnp.float32),
            jax.ShapeDtypeStruct((N_NODES, width), jnp.float32),
        ),
        mesh=mesh,
        scratch_types=[
            pltpu.VMEM((CH_ROWS, 128), jnp.int32),
            pltpu.VMEM((CH_ROWS, 128), jnp.int32),
            pltpu.VMEM((128, width), jnp.float32),
            pltpu.VMEM((128, width), jnp.float32),
            pltpu.VMEM_SHARED((N_NODES, width), jnp.float32),
            pltpu.SemaphoreType.DMA,
            pltpu.SemaphoreType.DMA,
        ],
        compiler_params=pltpu.CompilerParams(use_tc_tiling_on_sc=False),
    )(_sc_agg_body)
    return k(ht, gidx2, dst2, zeros_slab)


def _decode_body(a0_ref, b0_ref, a1_ref, b1_ref, ei_ref, ri_ref, td_ref,
                 rel_ref, w1_ref, w2_ref, w3_ref, b_ref, temb_ref,
                 static_out, y_out):
    p0 = a0_ref[...] + b0_ref[...]
    p1 = a1_ref[...] + b1_ref[...]
    deg = p1[:, DEG_COL:DEG_COL + 1]
    norm = jnp.where(deg > 0, 1.0 / jnp.maximum(deg, 1.0), 0.0)
    st = jnp.concatenate([p0, p1[:, :HIDDEN - W_A]], axis=1) * norm
    st = jnp.where(st >= 0, st, st * RRELU_SLOPE)
    static_out[...] = jnp.concatenate(
        [st, jnp.zeros((NUM_ENT_PAD - NUM_ENT, HIDDEN), jnp.float32)], axis=0)

    ei = ei_ref[...]
    acc = jnp.zeros((BATCH, HIDDEN), dtype=jnp.float32)
    chunk = 1000
    for k in range(NUM_ENT // chunk):
        iota = lax.broadcasted_iota(jnp.int32, (BATCH, chunk), 1) + k * chunk
        oh = (ei == iota).astype(jnp.float32)
        acc = acc + jnp.dot(oh, st[k * chunk:(k + 1) * chunk, :],
                            preferred_element_type=jnp.float32)
    ent = jnp.tanh(acc)

    ri = ri_ref[...]
    iota_r = lax.broadcasted_iota(jnp.int32, (BATCH, 230), 1)
    oh_r = (ri == iota_r).astype(jnp.float32)
    rel = jnp.dot(oh_r, rel_ref[...], preferred_element_type=jnp.float32)

    ti = td_ref[...] // 24
    iota_t = lax.broadcasted_iota(jnp.int32, (BATCH, 365), 1)
    oh_t = (ti == iota_t).astype(jnp.float32)
    tim = jnp.dot(oh_t, temb_ref[...], preferred_element_type=jnp.float32)

    x = (jnp.dot(ent, w1_ref[...], preferred_element_type=jnp.float32)
         + jnp.dot(rel, w2_ref[...], preferred_element_type=jnp.float32)
         + tim * w3_ref[...] + b_ref[...])
    y_out[...] = jnp.maximum(x, 0.0)


def _decode(a0, b0, a1, b1, ei, ri, td, rel_emb, w1, w2, w3, b, temb):
    return pl.pallas_call(
        _decode_body,
        out_shape=[
            jax.ShapeDtypeStruct((NUM_ENT_PAD, HIDDEN), jnp.float32),
            jax.ShapeDtypeStruct((BATCH, HIDDEN), jnp.float32),
        ],
    )(a0, b0, a1, b1, ei, ri, td, rel_emb, w1, w2, w3, b, temb)


_SC_BLK = 1024


def _score_body(y_ref, s_ref, out_ref):
    out_ref[...] = lax.dot_general(
        y_ref[...], s_ref[...], (((1,), (1,)), ((), ())),
        preferred_element_type=jnp.float32)


def _score(y, static):
    nb = NUM_ENT_PAD // _SC_BLK
    return pl.pallas_call(
        _score_body,
        grid=(nb,),
        in_specs=[
            pl.BlockSpec((BATCH, HIDDEN), lambda b: (0, 0)),
            pl.BlockSpec((_SC_BLK, HIDDEN), lambda b: (b, 0)),
        ],
        out_specs=pl.BlockSpec((BATCH, _SC_BLK), lambda b: (0, b)),
        out_shape=jax.ShapeDtypeStruct((BATCH, NUM_ENT_PAD), jnp.float32),
    )(y, static)


def kernel(ent_emb, attr_emb, rel_emb, rgcn_weight, dec_W, dec_b, time_emb,
           edge_index, edge_type, batch_data):
    f32 = jnp.float32
    h = jnp.concatenate([ent_emb, attr_emb], axis=0)
    w = rgcn_weight.reshape(NUM_TYPES, HIDDEN)

    h_a = h[:, :W_A]
    w_a = w[:, :W_A]
    ones_n = jnp.ones((N_NODES, 1), f32)
    pad_n = jnp.zeros((N_NODES, W_B - DEG_COL - 1), f32)
    h_b = jnp.concatenate([h[:, W_A:], ones_n, pad_n], axis=1)
    w_b = jnp.concatenate(
        [w[:, W_A:], jnp.ones((NUM_TYPES, 1), f32),
         jnp.zeros((NUM_TYPES, W_B - DEG_COL - 1), f32)], axis=1)

    src2 = edge_index[0].reshape(IDX_ROWS, 128)
    et2 = edge_type.reshape(IDX_ROWS, 128)
    dst2 = edge_index[1].reshape(IDX_ROWS, 128)

    gidx2 = _build_gidx(src2, et2)
    ht_a = _build_ht(h_a, w_a, W_A)
    ht_b = _build_ht(h_b, w_b, W_B)

    pa0, pb0 = _sc_agg(ht_a, gidx2, dst2,
                       jnp.zeros((ZS_LAST, W_A), f32), W_A)
    pa1, pb1 = _sc_agg(ht_b, gidx2, dst2,
                       jnp.zeros((ZS_LAST, W_B), f32), W_B)

    ei = batch_data[:, 0:1]
    ri = batch_data[:, 1:2]
    td = batch_data[:, 3:4]
    w1 = dec_W[0:HIDDEN]
    w2 = dec_W[HIDDEN:2 * HIDDEN]
    w3 = dec_W[2 * HIDDEN:2 * HIDDEN + 1]
    b = dec_b.reshape(1, HIDDEN)

    static, y = _decode(pa0[:NUM_ENT], pb0[:NUM_ENT],
                        pa1[:NUM_ENT], pb1[:NUM_ENT],
                        ei, ri, td, rel_emb, w1, w2, w3, b, time_emb)
    return _score(y, static)[:, :NUM_ENT]
